# Initial kernel scaffold; baseline (speedup 1.0000x reference)
import jax
import jax.numpy as jnp
from jax.experimental import pallas as pl


def kernel(input_ids, token_type_ids, word_embeddings, position_embeddings, gamma, beta):
    raise NotImplementedError("write your pallas kernel here")


# speedup vs baseline: 1.2711x; 1.2711x over previous
import functools

import jax
import jax.numpy as jnp
from jax import lax
from jax.experimental import pallas as pl
from jax.experimental.pallas import tpu as pltpu
from jax.experimental.pallas import tpu_sc as plsc

VOCAB = 100000
HIDDEN = 768
EPS = 1e-3

NUM_SC_CORES = 2
NUM_SC_SUBCORES = 16
NUM_WORKERS = NUM_SC_CORES * NUM_SC_SUBCORES

CHUNK = 64


def _sc_gather(table, idx_flat, num_rows):
    rows_per_worker = num_rows // NUM_WORKERS
    mesh = plsc.VectorSubcoreMesh(
        core_axis_name="c", subcore_axis_name="s",
        num_cores=NUM_SC_CORES, num_subcores=NUM_SC_SUBCORES)

    @functools.partial(
        pl.kernel,
        out_type=jax.ShapeDtypeStruct((num_rows, HIDDEN), jnp.float32),
        mesh=mesh,
        scratch_types=[
            pltpu.VMEM((rows_per_worker,), jnp.int32),
            pltpu.VMEM((CHUNK, HIDDEN), jnp.float32),
            pltpu.SemaphoreType.DMA,
        ],
    )
    def gather_kernel(table_hbm, idx_hbm, out_hbm, idx_v, rows_v, sem):
        wid = lax.axis_index("s") * NUM_SC_CORES + lax.axis_index("c")
        base = wid * rows_per_worker
        pltpu.sync_copy(idx_hbm.at[pl.ds(base, rows_per_worker)], idx_v)

        @pl.loop(0, rows_per_worker, step=CHUNK)
        def _(c):
            pltpu.async_copy(
                table_hbm.at[idx_v.at[pl.ds(c, CHUNK)]], rows_v, sem
            ).wait()
            pltpu.sync_copy(rows_v, out_hbm.at[pl.ds(base + c, CHUNK)])

    return gather_kernel(table, idx_flat)


def _tc_add_pos_layernorm(gathered, pos, gamma, beta, seq_len):
    num_rows = gathered.shape[0]
    blk = 512
    pos_blocks = seq_len // blk

    def body(x_ref, p_ref, g_ref, b_ref, o_ref):
        x = x_ref[...] + p_ref[...]
        mean = jnp.mean(x, axis=1, keepdims=True)
        xc = x - mean
        var = jnp.mean(xc * xc, axis=1, keepdims=True)
        o_ref[...] = (xc * lax.rsqrt(var + EPS)) * g_ref[...] + b_ref[...]

    return pl.pallas_call(
        body,
        grid=(num_rows // blk,),
        in_specs=[
            pl.BlockSpec((blk, HIDDEN), lambda i: (i, 0)),
            pl.BlockSpec((blk, HIDDEN), lambda i: (i % pos_blocks, 0)),
            pl.BlockSpec((1, HIDDEN), lambda i: (0, 0)),
            pl.BlockSpec((1, HIDDEN), lambda i: (0, 0)),
        ],
        out_specs=pl.BlockSpec((blk, HIDDEN), lambda i: (i, 0)),
        out_shape=jax.ShapeDtypeStruct((num_rows, HIDDEN), jnp.float32),
    )(gathered, pos, gamma, beta)


def kernel(input_ids, token_type_ids, word_embeddings, position_embeddings,
           gamma, beta):
    batch, seq = input_ids.shape
    num_rows = batch * seq
    idx_flat = input_ids.reshape(num_rows)
    gathered = _sc_gather(word_embeddings, idx_flat, num_rows)
    out = _tc_add_pos_layernorm(
        gathered,
        position_embeddings[:seq],
        gamma.reshape(1, HIDDEN),
        beta.reshape(1, HIDDEN),
        seq,
    )
    return out.reshape(batch, seq, HIDDEN)


# speedup vs baseline: 1.3380x; 1.0526x over previous
import functools

import jax
import jax.numpy as jnp
from jax import lax
from jax.experimental import pallas as pl
from jax.experimental.pallas import tpu as pltpu
from jax.experimental.pallas import tpu_sc as plsc

VOCAB = 100000
HIDDEN = 768
EPS = 1e-3

NUM_SC_CORES = 2
NUM_SC_SUBCORES = 16
NUM_WORKERS = NUM_SC_CORES * NUM_SC_SUBCORES

CHUNK = 64


def _sc_gather(table, idx_flat, num_rows):
    rows_per_worker = num_rows // NUM_WORKERS
    mesh = plsc.VectorSubcoreMesh(
        core_axis_name="c", subcore_axis_name="s",
        num_cores=NUM_SC_CORES, num_subcores=NUM_SC_SUBCORES)

    n_chunks = rows_per_worker // CHUNK

    @functools.partial(
        pl.kernel,
        out_type=jax.ShapeDtypeStruct((num_rows, HIDDEN), jnp.float32),
        mesh=mesh,
        scratch_types=[
            pltpu.VMEM((rows_per_worker,), jnp.int32),
            pltpu.VMEM((CHUNK, HIDDEN), jnp.float32),
            pltpu.VMEM((CHUNK, HIDDEN), jnp.float32),
            pltpu.SemaphoreType.DMA,
            pltpu.SemaphoreType.DMA,
            pltpu.SemaphoreType.DMA,
            pltpu.SemaphoreType.DMA,
        ],
    )
    def gather_kernel(table_hbm, idx_hbm, out_hbm, idx_v, rows_a, rows_b,
                      gsem_a, gsem_b, wsem_a, wsem_b):
        wid = lax.axis_index("s") * NUM_SC_CORES + lax.axis_index("c")
        base = wid * rows_per_worker
        pltpu.sync_copy(idx_hbm.at[pl.ds(base, rows_per_worker)], idx_v)

        bufs = (rows_a, rows_b)
        gsems = (gsem_a, gsem_b)
        wsems = (wsem_a, wsem_b)

        def start_gather(k):
            return pltpu.async_copy(
                table_hbm.at[idx_v.at[pl.ds(k * CHUNK, CHUNK)]],
                bufs[k % 2], gsems[k % 2])

        def start_write(k):
            return pltpu.async_copy(
                bufs[k % 2], out_hbm.at[pl.ds(base + k * CHUNK, CHUNK)],
                wsems[k % 2])

        g = {0: start_gather(0)}
        if n_chunks > 1:
            g[1] = start_gather(1)
        w = {}
        for k in range(n_chunks):
            g[k].wait()
            w[k] = start_write(k)
            if k + 2 < n_chunks:
                w[k].wait()
                g[k + 2] = start_gather(k + 2)
        for k in range(max(0, n_chunks - 2), n_chunks):
            w[k].wait()

    return gather_kernel(table, idx_flat)


def _tc_add_pos_layernorm(gathered, pos, gamma, beta, seq_len):
    num_rows = gathered.shape[0]
    blk = 512
    pos_blocks = seq_len // blk
    batch = num_rows // seq_len

    def body(x_ref, p_ref, g_ref, b_ref, o_ref):
        x = x_ref[...] + p_ref[...]
        mean = jnp.mean(x, axis=1, keepdims=True)
        xc = x - mean
        var = jnp.mean(xc * xc, axis=1, keepdims=True)
        o_ref[...] = (xc * lax.rsqrt(var + EPS)) * g_ref[...] + b_ref[...]

    return pl.pallas_call(
        body,
        grid=(pos_blocks, batch),
        in_specs=[
            pl.BlockSpec((blk, HIDDEN), lambda s, b: (b * pos_blocks + s, 0)),
            pl.BlockSpec((blk, HIDDEN), lambda s, b: (s, 0)),
            pl.BlockSpec((1, HIDDEN), lambda s, b: (0, 0)),
            pl.BlockSpec((1, HIDDEN), lambda s, b: (0, 0)),
        ],
        out_specs=pl.BlockSpec((blk, HIDDEN), lambda s, b: (b * pos_blocks + s, 0)),
        out_shape=jax.ShapeDtypeStruct((num_rows, HIDDEN), jnp.float32),
    )(gathered, pos, gamma, beta)


def kernel(input_ids, token_type_ids, word_embeddings, position_embeddings,
           gamma, beta):
    batch, seq = input_ids.shape
    num_rows = batch * seq
    idx_flat = input_ids.reshape(num_rows)
    gathered = _sc_gather(word_embeddings, idx_flat, num_rows)
    out = _tc_add_pos_layernorm(
        gathered,
        position_embeddings[:seq],
        gamma.reshape(1, HIDDEN),
        beta.reshape(1, HIDDEN),
        seq,
    )
    return out.reshape(batch, seq, HIDDEN)


# speedup vs baseline: 1.3434x; 1.0041x over previous
import functools

import jax
import jax.numpy as jnp
from jax import lax
from jax.experimental import pallas as pl
from jax.experimental.pallas import tpu as pltpu
from jax.experimental.pallas import tpu_sc as plsc

VOCAB = 100000
HIDDEN = 768
EPS = 1e-3

NUM_SC_CORES = 2
NUM_SC_SUBCORES = 16
NUM_WORKERS = NUM_SC_CORES * NUM_SC_SUBCORES

CHUNK = 64


def _sc_gather(table, input_ids):
    batch, seq = input_ids.shape
    num_rows = batch * seq
    rows_per_worker = num_rows // NUM_WORKERS
    workers_per_batch = seq // rows_per_worker
    mesh = plsc.VectorSubcoreMesh(
        core_axis_name="c", subcore_axis_name="s",
        num_cores=NUM_SC_CORES, num_subcores=NUM_SC_SUBCORES)

    n_chunks = rows_per_worker // CHUNK

    @functools.partial(
        pl.kernel,
        out_type=jax.ShapeDtypeStruct((num_rows, HIDDEN), jnp.float32),
        mesh=mesh,
        scratch_types=[
            pltpu.VMEM((rows_per_worker,), jnp.int32),
            pltpu.VMEM((CHUNK, HIDDEN), jnp.float32),
            pltpu.VMEM((CHUNK, HIDDEN), jnp.float32),
            pltpu.SemaphoreType.DMA,
            pltpu.SemaphoreType.DMA,
            pltpu.SemaphoreType.DMA,
            pltpu.SemaphoreType.DMA,
        ],
    )
    def gather_kernel(table_hbm, idx_hbm, out_hbm, idx_v, rows_a, rows_b,
                      gsem_a, gsem_b, wsem_a, wsem_b):
        wid = lax.axis_index("s") * NUM_SC_CORES + lax.axis_index("c")
        base = wid * rows_per_worker
        b = wid // workers_per_batch
        soff = (wid % workers_per_batch) * rows_per_worker
        pltpu.sync_copy(idx_hbm.at[b, pl.ds(soff, rows_per_worker)], idx_v)

        bufs = (rows_a, rows_b)
        gsems = (gsem_a, gsem_b)
        wsems = (wsem_a, wsem_b)

        def start_gather(k):
            return pltpu.async_copy(
                table_hbm.at[idx_v.at[pl.ds(k * CHUNK, CHUNK)]],
                bufs[k % 2], gsems[k % 2])

        def start_write(k):
            return pltpu.async_copy(
                bufs[k % 2], out_hbm.at[pl.ds(base + k * CHUNK, CHUNK)],
                wsems[k % 2])

        g = {0: start_gather(0)}
        if n_chunks > 1:
            g[1] = start_gather(1)
        w = {}
        for k in range(n_chunks):
            g[k].wait()
            w[k] = start_write(k)
            if k + 2 < n_chunks:
                w[k].wait()
                g[k + 2] = start_gather(k + 2)
        for k in range(max(0, n_chunks - 2), n_chunks):
            w[k].wait()

    return gather_kernel(table, input_ids)


def _tc_add_pos_layernorm(gathered, pos, gamma, beta, seq_len):
    num_rows = gathered.shape[0]
    blk = 512
    pos_blocks = seq_len // blk
    batch = num_rows // seq_len

    def body(x_ref, p_ref, g_ref, b_ref, o_ref):
        x = x_ref[...] + p_ref[...]
        mean = jnp.mean(x, axis=1, keepdims=True)
        xc = x - mean
        var = jnp.mean(xc * xc, axis=1, keepdims=True)
        o_ref[0] = (xc * lax.rsqrt(var + EPS)) * g_ref[...] + b_ref[...]

    return pl.pallas_call(
        body,
        grid=(pos_blocks, batch),
        in_specs=[
            pl.BlockSpec((blk, HIDDEN), lambda s, b: (b * pos_blocks + s, 0)),
            pl.BlockSpec((blk, HIDDEN), lambda s, b: (s, 0)),
            pl.BlockSpec((1, HIDDEN), lambda s, b: (0, 0)),
            pl.BlockSpec((1, HIDDEN), lambda s, b: (0, 0)),
        ],
        out_specs=pl.BlockSpec((1, blk, HIDDEN), lambda s, b: (b, s, 0)),
        out_shape=jax.ShapeDtypeStruct((batch, seq_len, HIDDEN), jnp.float32),
    )(gathered, pos, gamma, beta)


def kernel(input_ids, token_type_ids, word_embeddings, position_embeddings,
           gamma, beta):
    batch, seq = input_ids.shape
    gathered = _sc_gather(word_embeddings, input_ids)
    return _tc_add_pos_layernorm(
        gathered,
        position_embeddings[:seq],
        gamma.reshape(1, HIDDEN),
        beta.reshape(1, HIDDEN),
        seq,
    )


# speedup vs baseline: 1.4175x; 1.0552x over previous
import functools

import jax
import jax.numpy as jnp
from jax import lax
from jax.experimental import pallas as pl
from jax.experimental.pallas import tpu as pltpu
from jax.experimental.pallas import tpu_sc as plsc

VOCAB = 100000
HIDDEN = 768
EPS = 1e-3

NUM_SC_CORES = 2
NUM_SC_SUBCORES = 16
NUM_WORKERS = NUM_SC_CORES * NUM_SC_SUBCORES

CHUNK = 64


def _sc_gather(table, input_ids):
    batch, seq = input_ids.shape
    num_rows = batch * seq
    rows_per_worker = num_rows // NUM_WORKERS
    workers_per_batch = seq // rows_per_worker
    mesh = plsc.VectorSubcoreMesh(
        core_axis_name="c", subcore_axis_name="s",
        num_cores=NUM_SC_CORES, num_subcores=NUM_SC_SUBCORES)

    n_chunks = rows_per_worker // CHUNK

    @functools.partial(
        pl.kernel,
        out_type=jax.ShapeDtypeStruct((num_rows, HIDDEN), jnp.float32),
        mesh=mesh,
        scratch_types=[
            pltpu.VMEM((rows_per_worker,), jnp.int32),
            pltpu.VMEM((CHUNK, HIDDEN), jnp.float32),
            pltpu.VMEM((CHUNK, HIDDEN), jnp.float32),
            pltpu.SemaphoreType.DMA,
            pltpu.SemaphoreType.DMA,
            pltpu.SemaphoreType.DMA,
            pltpu.SemaphoreType.DMA,
        ],
    )
    def gather_kernel(table_hbm, idx_hbm, out_hbm, idx_v, rows_a, rows_b,
                      gsem_a, gsem_b, wsem_a, wsem_b):
        wid = lax.axis_index("s") * NUM_SC_CORES + lax.axis_index("c")
        base = wid * rows_per_worker
        b = wid // workers_per_batch
        soff = (wid % workers_per_batch) * rows_per_worker
        pltpu.sync_copy(idx_hbm.at[b, pl.ds(soff, rows_per_worker)], idx_v)

        bufs = (rows_a, rows_b)
        gsems = (gsem_a, gsem_b)
        wsems = (wsem_a, wsem_b)

        def start_gather(k):
            return pltpu.async_copy(
                table_hbm.at[idx_v.at[pl.ds(k * CHUNK, CHUNK)]],
                bufs[k % 2], gsems[k % 2])

        def start_write(k):
            return pltpu.async_copy(
                bufs[k % 2], out_hbm.at[pl.ds(base + k * CHUNK, CHUNK)],
                wsems[k % 2])

        g = {0: start_gather(0)}
        if n_chunks > 1:
            g[1] = start_gather(1)
        w = {}
        for k in range(n_chunks):
            g[k].wait()
            w[k] = start_write(k)
            if k + 2 < n_chunks:
                w[k].wait()
                g[k + 2] = start_gather(k + 2)
        for k in range(max(0, n_chunks - 2), n_chunks):
            w[k].wait()

    return gather_kernel(table, input_ids)


def _tc_add_pos_layernorm(gathered, pos, gamma, beta, seq_len):
    num_rows = gathered.shape[0]
    blk = 1024
    pos_blocks = seq_len // blk
    batch = num_rows // seq_len

    def body(x_ref, p_ref, g_ref, b_ref, o_ref):
        x = x_ref[...] + p_ref[...]
        mean = jnp.mean(x, axis=1, keepdims=True)
        xc = x - mean
        var = jnp.mean(xc * xc, axis=1, keepdims=True)
        o_ref[0] = (xc * lax.rsqrt(var + EPS)) * g_ref[...] + b_ref[...]

    return pl.pallas_call(
        body,
        grid=(pos_blocks, batch),
        in_specs=[
            pl.BlockSpec((blk, HIDDEN), lambda s, b: (b * pos_blocks + s, 0)),
            pl.BlockSpec((blk, HIDDEN), lambda s, b: (s, 0)),
            pl.BlockSpec((1, HIDDEN), lambda s, b: (0, 0)),
            pl.BlockSpec((1, HIDDEN), lambda s, b: (0, 0)),
        ],
        out_specs=pl.BlockSpec((1, blk, HIDDEN), lambda s, b: (b, s, 0)),
        out_shape=jax.ShapeDtypeStruct((batch, seq_len, HIDDEN), jnp.float32),
    )(gathered, pos, gamma, beta)


def kernel(input_ids, token_type_ids, word_embeddings, position_embeddings,
           gamma, beta):
    batch, seq = input_ids.shape
    gathered = _sc_gather(word_embeddings, input_ids)
    return _tc_add_pos_layernorm(
        gathered,
        position_embeddings[:seq],
        gamma.reshape(1, HIDDEN),
        beta.reshape(1, HIDDEN),
        seq,
    )


# speedup vs baseline: 1.4437x; 1.0184x over previous
import functools

import jax
import jax.numpy as jnp
from jax import lax
from jax.experimental import pallas as pl
from jax.experimental.pallas import tpu as pltpu
from jax.experimental.pallas import tpu_sc as plsc

VOCAB = 100000
HIDDEN = 768
EPS = 1e-3

NUM_SC_CORES = 2
NUM_SC_SUBCORES = 16
NUM_WORKERS = NUM_SC_CORES * NUM_SC_SUBCORES

CHUNK = 64


def _sc_gather(table, input_ids):
    batch, seq = input_ids.shape
    num_rows = batch * seq
    rows_per_worker = num_rows // NUM_WORKERS
    workers_per_batch = seq // rows_per_worker
    mesh = plsc.VectorSubcoreMesh(
        core_axis_name="c", subcore_axis_name="s",
        num_cores=NUM_SC_CORES, num_subcores=NUM_SC_SUBCORES)

    n_chunks = rows_per_worker // CHUNK

    @functools.partial(
        pl.kernel,
        out_type=jax.ShapeDtypeStruct((num_rows, HIDDEN), jnp.float32),
        mesh=mesh,
        scratch_types=[
            pltpu.VMEM((rows_per_worker,), jnp.int32),
            pltpu.VMEM((CHUNK, HIDDEN), jnp.float32),
            pltpu.VMEM((CHUNK, HIDDEN), jnp.float32),
            pltpu.SemaphoreType.DMA,
            pltpu.SemaphoreType.DMA,
            pltpu.SemaphoreType.DMA,
            pltpu.SemaphoreType.DMA,
        ],
    )
    def gather_kernel(table_hbm, idx_hbm, out_hbm, idx_v, rows_a, rows_b,
                      gsem_a, gsem_b, wsem_a, wsem_b):
        wid = lax.axis_index("s") * NUM_SC_CORES + lax.axis_index("c")
        base = wid * rows_per_worker
        b = wid // workers_per_batch
        soff = (wid % workers_per_batch) * rows_per_worker
        pltpu.sync_copy(idx_hbm.at[b, pl.ds(soff, rows_per_worker)], idx_v)

        bufs = (rows_a, rows_b)
        gsems = (gsem_a, gsem_b)
        wsems = (wsem_a, wsem_b)

        def start_gather(k):
            return pltpu.async_copy(
                table_hbm.at[idx_v.at[pl.ds(k * CHUNK, CHUNK)]],
                bufs[k % 2], gsems[k % 2])

        def start_write(k):
            return pltpu.async_copy(
                bufs[k % 2], out_hbm.at[pl.ds(base + k * CHUNK, CHUNK)],
                wsems[k % 2])

        g = {0: start_gather(0)}
        if n_chunks > 1:
            g[1] = start_gather(1)
        w = {}
        for k in range(n_chunks):
            g[k].wait()
            w[k] = start_write(k)
            if k + 2 < n_chunks:
                w[k].wait()
                g[k + 2] = start_gather(k + 2)
        for k in range(max(0, n_chunks - 2), n_chunks):
            w[k].wait()

    return gather_kernel(table, input_ids)


def _tc_add_pos_layernorm(gathered, pos, gamma, beta, seq_len):
    num_rows = gathered.shape[0]
    blk = 2048
    pos_blocks = seq_len // blk
    batch = num_rows // seq_len

    def body(x_ref, p_ref, g_ref, b_ref, o_ref):
        x = x_ref[...] + p_ref[...]
        mean = jnp.mean(x, axis=1, keepdims=True)
        xc = x - mean
        var = jnp.mean(xc * xc, axis=1, keepdims=True)
        o_ref[0] = (xc * lax.rsqrt(var + EPS)) * g_ref[...] + b_ref[...]

    return pl.pallas_call(
        body,
        grid=(pos_blocks, batch),
        in_specs=[
            pl.BlockSpec((blk, HIDDEN), lambda s, b: (b * pos_blocks + s, 0)),
            pl.BlockSpec((blk, HIDDEN), lambda s, b: (s, 0)),
            pl.BlockSpec((1, HIDDEN), lambda s, b: (0, 0)),
            pl.BlockSpec((1, HIDDEN), lambda s, b: (0, 0)),
        ],
        out_specs=pl.BlockSpec((1, blk, HIDDEN), lambda s, b: (b, s, 0)),
        out_shape=jax.ShapeDtypeStruct((batch, seq_len, HIDDEN), jnp.float32),
    )(gathered, pos, gamma, beta)


def kernel(input_ids, token_type_ids, word_embeddings, position_embeddings,
           gamma, beta):
    batch, seq = input_ids.shape
    gathered = _sc_gather(word_embeddings, input_ids)
    return _tc_add_pos_layernorm(
        gathered,
        position_embeddings[:seq],
        gamma.reshape(1, HIDDEN),
        beta.reshape(1, HIDDEN),
        seq,
    )
